# baseline (speedup 1.0000x reference)
import functools
import jax
import jax.numpy as jnp
from jax import lax
from jax.experimental import pallas as pl
from jax.experimental.pallas import tpu as pltpu
from jax.experimental.pallas import tpu_sc as plsc

_MARGIN = 0.2
_B = 4096
_D = 1024
_NCLS = 128
_BS = 1024
_NBLK = _B // _BS


def _lane_cumsum(x):
    n = x.shape[1]
    k = 1
    while k < n:
        shifted = jnp.concatenate(
            [jnp.zeros((x.shape[0], k), x.dtype), x[:, : n - k]], axis=1)
        x = x + shifted
        k *= 2
    return x


def _gat(vals, onehot):
    return jnp.sum(jnp.where(onehot, vals, 0), axis=0, keepdims=True)


def _mining_body(c1_ref, c2_ref,
                 j3_ref, t3_ref, sel3_ref, j4_ref, t4_ref, sel4_ref,
                 f1a_ref, f1b_ref, f2a_ref, f2b_ref):
    c1 = c1_ref[...]
    c2 = c2_ref[...]
    cid = lax.broadcasted_iota(jnp.int32, (_NCLS, _B), 0)
    pos = lax.broadcasted_iota(jnp.int32, (_NCLS, _B), 1)
    eq1 = c1 == cid
    eq2 = c2 == cid
    h1 = jnp.sum(eq1.astype(jnp.int32), axis=1, keepdims=True)
    h2 = jnp.sum(eq2.astype(jnp.int32), axis=1, keepdims=True)
    f1a = jnp.min(jnp.where(eq1, pos, _B), axis=1, keepdims=True)
    f1b = jnp.min(jnp.where(eq1 & (pos != f1a), pos, _B), axis=1, keepdims=True)
    f2a = jnp.min(jnp.where(eq2, pos, _B), axis=1, keepdims=True)
    f2b = jnp.min(jnp.where(eq2 & (pos != f2a), pos, _B), axis=1, keepdims=True)
    cum1 = _lane_cumsum(jnp.where(eq1, 0, 1))
    cum2 = _lane_cumsum(jnp.where(eq2, 0, 1))
    rpos = lax.broadcasted_iota(jnp.int32, (1, _B), 1)

    k3 = _B - jnp.max(_gat(h1, eq2)) + 1
    t3class = jnp.min(jnp.where(cum1 == k3, pos, _B), axis=1, keepdims=True)
    t3_ref[...] = _gat(t3class, eq2)
    f1a_at = _gat(f1a, eq2)
    sel3 = f1a_at == rpos
    sel3_ref[...] = sel3.astype(jnp.int32)
    j3_ref[...] = jnp.where(sel3, _gat(f1b, eq2), f1a_at)

    k4 = _B - jnp.max(_gat(h2, eq1)) + 1
    t4class = jnp.min(jnp.where(cum2 == k4, pos, _B), axis=1, keepdims=True)
    t4_ref[...] = _gat(t4class, eq1)
    f2a_at = _gat(f2a, eq1)
    sel4 = f2a_at == rpos
    sel4_ref[...] = sel4.astype(jnp.int32)
    j4_ref[...] = jnp.where(sel4, _gat(f2b, eq1), f2a_at)

    f1a_ref[...] = jnp.minimum(f1a, _B - 1)
    f1b_ref[...] = jnp.minimum(f1b, _B - 1)
    f2a_ref[...] = jnp.minimum(f2a, _B - 1)
    f2b_ref[...] = jnp.minimum(f2b, _B - 1)


def _mining(c1, c2):
    row = jax.ShapeDtypeStruct((1, _B), jnp.int32)
    col = jax.ShapeDtypeStruct((_NCLS, 1), jnp.int32)
    return pl.pallas_call(
        _mining_body,
        out_shape=[row] * 6 + [col] * 4,
    )(c1, c2)


_SC_CORES = 2
_SC_SUBCORES = 16
_NW = _SC_CORES * _SC_SUBCORES
_CH = _NCLS // 8


_sc_gather_built = None


def _build_sc_gather():
    @functools.partial(
        pl.kernel,
        mesh=plsc.VectorSubcoreMesh(core_axis_name="c", subcore_axis_name="s",
                                    num_cores=_SC_CORES,
                                    num_subcores=_SC_SUBCORES),
        out_type=[jax.ShapeDtypeStruct((_NCLS, _D), jnp.float32)] * 4,
        scratch_types=[pltpu.VMEM((_CH,), jnp.int32),
                       pltpu.VMEM((_CH, _D), jnp.float32),
                       pltpu.SemaphoreType.DMA],
    )
    def sc_gather(x2_hbm, x1_hbm, f1a_hbm, f1b_hbm, f2a_hbm, f2b_hbm,
                  u1_hbm, u2_hbm, v1_hbm, v2_hbm, idx_v, rows_v, sem):
        wid = lax.axis_index("s") * _SC_CORES + lax.axis_index("c")
        tid = wid // 8
        off = (wid % 8) * _CH
        jobs = ((x2_hbm, f1a_hbm, u1_hbm), (x2_hbm, f1b_hbm, u2_hbm),
                (x1_hbm, f2a_hbm, v1_hbm), (x1_hbm, f2b_hbm, v2_hbm))
        for t, (table, idx_hbm, out) in enumerate(jobs):
            @pl.when(tid == t)
            def _(table=table, idx_hbm=idx_hbm, out=out):
                pltpu.sync_copy(idx_hbm.at[pl.ds(off, _CH)], idx_v)
                pltpu.async_copy(table.at[idx_v], rows_v, sem).wait()
                pltpu.sync_copy(rows_v, out.at[pl.ds(off, _CH)])

    return sc_gather


def _sc_gather(*args):
    global _sc_gather_built
    if _sc_gather_built is None:
        _sc_gather_built = _build_sc_gather()
    return _sc_gather_built(*args)


def _fused_body(x1_ref, x2_ref, x1c_ref, u1_ref, u2_ref, v1_ref, v2_ref,
                j3_ref, sel3_ref, t3_ref, cls2_ref,
                j4_ref, sel4_ref, t4_ref, cls1_ref,
                s1_ref, n1_ref, s2_ref, n2_ref,
                s3_ref, n3_ref, s4_ref, n4_ref,
                coldat_ref, rowdat_ref):
    ri = pl.program_id(0)
    ci = pl.program_id(1)
    x1 = x1_ref[...]
    x2 = x2_ref[...]
    dims = (((1,), (1,)), ((), ()))
    ohc = lax.broadcasted_iota(jnp.int32, (_BS, _NCLS), 1)

    @pl.when(ri == 0)
    def _():
        diag_cb = 1.0 - jnp.sum(x1c_ref[...] * x2, axis=1)
        qa = lax.dot_general(x2, v1_ref[...], dims,
                             preferred_element_type=jnp.float32)
        qb = lax.dot_general(x2, v2_ref[...], dims,
                             preferred_element_type=jnp.float32)
        oh1 = cls1_ref[0, 0, :][:, None] == ohc
        q = jnp.where(sel4_ref[0, 0, :] == 1,
                      jnp.sum(jnp.where(oh1, qb, 0.0), axis=1),
                      jnp.sum(jnp.where(oh1, qa, 0.0), axis=1))
        coldat_ref[ci, 0, :] = diag_cb
        coldat_ref[ci, 1, :] = jnp.where(j4_ref[0, 0, :] < _B, 1.0 - q, 0.0)

    @pl.when(ci == 0)
    def _():
        pa = lax.dot_general(x1, u1_ref[...], dims,
                             preferred_element_type=jnp.float32)
        pb = lax.dot_general(x1, u2_ref[...], dims,
                             preferred_element_type=jnp.float32)
        oh2 = cls2_ref[0, 0, :][:, None] == ohc
        p = jnp.where(sel3_ref[0, 0, :] == 1,
                      jnp.sum(jnp.where(oh2, pb, 0.0), axis=1),
                      jnp.sum(jnp.where(oh2, pa, 0.0), axis=1))
        rowdat_ref[0, :] = jnp.where(j3_ref[0, 0, :] < _B, 1.0 - p, 0.0)

    g2 = (1.0 - _MARGIN) - lax.dot_general(
        x1, x2, dims, preferred_element_type=jnp.float32)

    io = lax.broadcasted_iota(jnp.int32, (_BS, _BS), 0)
    jo = lax.broadcasted_iota(jnp.int32, (_BS, _BS), 1)
    offd = (ri != ci) | (io != jo)

    def term(keep, v):
        s = jnp.sum(jnp.where(keep, v, 0.0))
        n = jnp.sum(jnp.where(keep & (v > 0), 1.0, 0.0))
        return s, n

    v1 = jnp.maximum(coldat_ref[ri, 0, :][:, None] - g2, 0.0)
    s1, n1 = term(offd, v1)
    v2 = jnp.maximum(coldat_ref[ci, 0, :][None, :] - g2, 0.0)
    s2, n2 = term(offd, v2)
    anti = cls1_ref[0, 0, :][None, :] != cls2_ref[0, 0, :][:, None]
    th3 = t3_ref[0, 0, :] - ci * _BS
    th4 = t4_ref[0, 0, :] - ri * _BS
    kept3 = anti & (jo < th3[:, None])
    v3 = jnp.maximum(rowdat_ref[0, :][:, None] - g2, 0.0)
    s3, n3 = term(kept3, v3)
    kept4 = anti & (io < th4[None, :])
    v4 = jnp.maximum(coldat_ref[ci, 1, :][None, :] - g2, 0.0)
    s4, n4 = term(kept4, v4)

    first = (ri == 0) & (ci == 0)
    for ref, val in ((s1_ref, s1), (n1_ref, n1), (s2_ref, s2), (n2_ref, n2),
                     (s3_ref, s3), (n3_ref, n3), (s4_ref, s4), (n4_ref, n4)):
        @pl.when(first)
        def _(ref=ref):
            ref[0, 0] = jnp.zeros((), jnp.float32)
        ref[0, 0] += val


def _fused(x1, x2, u1, u2, v1, v2,
           j3, sel3, t3, cls2, j4, sel4, t4, cls1):
    rmat = pl.BlockSpec((_BS, _D), lambda r, c: (r, 0))
    cmat = pl.BlockSpec((_BS, _D), lambda r, c: (c, 0))
    c0mat = pl.BlockSpec((_BS, _D),
                         lambda r, c: (jnp.where(r == 0, c, 0), 0))
    rep = pl.BlockSpec((_NCLS, _D), lambda r, c: (0, 0))
    rvec = pl.BlockSpec((1, 1, _BS), lambda r, c: (r, 0, 0))
    cvec = pl.BlockSpec((1, 1, _BS), lambda r, c: (c, 0, 0))
    acc = pl.BlockSpec(memory_space=pltpu.SMEM)
    out = jax.ShapeDtypeStruct((1, 1), jnp.float32)
    return pl.pallas_call(
        _fused_body,
        grid=(_NBLK, _NBLK),
        in_specs=[rmat, cmat, c0mat, rep, rep, rep, rep,
                  rvec, rvec, rvec, rvec, cvec, cvec, cvec, cvec],
        out_specs=[acc] * 8,
        out_shape=[out] * 8,
        scratch_shapes=[pltpu.VMEM((_NBLK, 2, _BS), jnp.float32),
                        pltpu.VMEM((2, _BS), jnp.float32)],
        compiler_params=pltpu.CompilerParams(
            dimension_semantics=("arbitrary", "arbitrary")),
    )(x1, x2, x1, u1, u2, v1, v2, j3, sel3, t3, cls2, j4, sel4, t4, cls1)


def kernel(input1, input2, class1, class2):
    c1 = class1.astype(jnp.int32).reshape(1, _B)
    c2 = class2.astype(jnp.int32).reshape(1, _B)
    (j3, t3, sel3, j4, t4, sel4,
     f1a, f1b, f2a, f2b) = _mining(c1, c2)
    u1, u2, v1, v2 = _sc_gather(input2, input1,
                                f1a.reshape(_NCLS), f1b.reshape(_NCLS),
                                f2a.reshape(_NCLS), f2b.reshape(_NCLS))
    blk = (_NBLK, 1, _BS)
    s1, n1, s2, n2, s3, n3, s4, n4 = _fused(
        input1, input2, u1, u2, v1, v2,
        j3.reshape(blk), sel3.reshape(blk), t3.reshape(blk), c2.reshape(blk),
        j4.reshape(blk), sel4.reshape(blk), t4.reshape(blk), c1.reshape(blk))
    loss = (s1[0, 0] / n1[0, 0] + s2[0, 0] / n2[0, 0]
            + s3[0, 0] / n3[0, 0] + s4[0, 0] / n4[0, 0])
    return loss.reshape(1)


# speedup vs baseline: 1.0291x; 1.0291x over previous
import functools
import jax
import jax.numpy as jnp
from jax import lax
from jax.experimental import pallas as pl
from jax.experimental.pallas import tpu as pltpu
from jax.experimental.pallas import tpu_sc as plsc

_MARGIN = 0.2
_B = 4096
_D = 1024
_NCLS = 128
_BS = 1024
_NBLK = _B // _BS
_BR = 2048
_BC = 512
_NBR = _B // _BR
_NBC = _B // _BC


def _lane_cumsum(x):
    n = x.shape[1]
    k = 1
    while k < n:
        shifted = jnp.concatenate(
            [jnp.zeros((x.shape[0], k), x.dtype), x[:, : n - k]], axis=1)
        x = x + shifted
        k *= 2
    return x


def _gat(vals, onehot):
    return jnp.sum(jnp.where(onehot, vals, 0), axis=0, keepdims=True)


def _mining_body(c1_ref, c2_ref,
                 j3_ref, t3_ref, sel3_ref, j4_ref, t4_ref, sel4_ref,
                 f1a_ref, f1b_ref, f2a_ref, f2b_ref):
    c1 = c1_ref[...]
    c2 = c2_ref[...]
    cid = lax.broadcasted_iota(jnp.int32, (_NCLS, _B), 0)
    pos = lax.broadcasted_iota(jnp.int32, (_NCLS, _B), 1)
    eq1 = c1 == cid
    eq2 = c2 == cid
    h1 = jnp.sum(eq1.astype(jnp.int32), axis=1, keepdims=True)
    h2 = jnp.sum(eq2.astype(jnp.int32), axis=1, keepdims=True)
    f1a = jnp.min(jnp.where(eq1, pos, _B), axis=1, keepdims=True)
    f1b = jnp.min(jnp.where(eq1 & (pos != f1a), pos, _B), axis=1, keepdims=True)
    f2a = jnp.min(jnp.where(eq2, pos, _B), axis=1, keepdims=True)
    f2b = jnp.min(jnp.where(eq2 & (pos != f2a), pos, _B), axis=1, keepdims=True)
    cum1 = _lane_cumsum(jnp.where(eq1, 0, 1))
    cum2 = _lane_cumsum(jnp.where(eq2, 0, 1))
    rpos = lax.broadcasted_iota(jnp.int32, (1, _B), 1)

    k3 = _B - jnp.max(_gat(h1, eq2)) + 1
    t3class = jnp.min(jnp.where(cum1 == k3, pos, _B), axis=1, keepdims=True)
    t3_ref[...] = _gat(t3class, eq2)
    f1a_at = _gat(f1a, eq2)
    sel3 = f1a_at == rpos
    sel3_ref[...] = sel3.astype(jnp.int32)
    j3_ref[...] = jnp.where(sel3, _gat(f1b, eq2), f1a_at)

    k4 = _B - jnp.max(_gat(h2, eq1)) + 1
    t4class = jnp.min(jnp.where(cum2 == k4, pos, _B), axis=1, keepdims=True)
    t4_ref[...] = _gat(t4class, eq1)
    f2a_at = _gat(f2a, eq1)
    sel4 = f2a_at == rpos
    sel4_ref[...] = sel4.astype(jnp.int32)
    j4_ref[...] = jnp.where(sel4, _gat(f2b, eq1), f2a_at)

    f1a_ref[...] = jnp.minimum(f1a, _B - 1)
    f1b_ref[...] = jnp.minimum(f1b, _B - 1)
    f2a_ref[...] = jnp.minimum(f2a, _B - 1)
    f2b_ref[...] = jnp.minimum(f2b, _B - 1)


def _mining(c1, c2):
    row = jax.ShapeDtypeStruct((1, _B), jnp.int32)
    col = jax.ShapeDtypeStruct((_NCLS, 1), jnp.int32)
    return pl.pallas_call(
        _mining_body,
        out_shape=[row] * 6 + [col] * 4,
    )(c1, c2)


_SC_CORES = 2
_SC_SUBCORES = 16
_NW = _SC_CORES * _SC_SUBCORES
_CH = _NCLS // 8


_sc_gather_built = None


def _build_sc_gather():
    @functools.partial(
        pl.kernel,
        mesh=plsc.VectorSubcoreMesh(core_axis_name="c", subcore_axis_name="s",
                                    num_cores=_SC_CORES,
                                    num_subcores=_SC_SUBCORES),
        out_type=[jax.ShapeDtypeStruct((_NCLS, _D), jnp.float32)] * 4,
        scratch_types=[pltpu.VMEM((_CH,), jnp.int32),
                       pltpu.VMEM((_CH, _D), jnp.float32),
                       pltpu.SemaphoreType.DMA],
    )
    def sc_gather(x2_hbm, x1_hbm, f1a_hbm, f1b_hbm, f2a_hbm, f2b_hbm,
                  u1_hbm, u2_hbm, v1_hbm, v2_hbm, idx_v, rows_v, sem):
        wid = lax.axis_index("s") * _SC_CORES + lax.axis_index("c")
        tid = wid // 8
        off = (wid % 8) * _CH
        jobs = ((x2_hbm, f1a_hbm, u1_hbm), (x2_hbm, f1b_hbm, u2_hbm),
                (x1_hbm, f2a_hbm, v1_hbm), (x1_hbm, f2b_hbm, v2_hbm))
        for t, (table, idx_hbm, out) in enumerate(jobs):
            @pl.when(tid == t)
            def _(table=table, idx_hbm=idx_hbm, out=out):
                pltpu.sync_copy(idx_hbm.at[pl.ds(off, _CH)], idx_v)
                pltpu.async_copy(table.at[idx_v], rows_v, sem).wait()
                pltpu.sync_copy(rows_v, out.at[pl.ds(off, _CH)])

    return sc_gather


def _sc_gather(*args):
    global _sc_gather_built
    if _sc_gather_built is None:
        _sc_gather_built = _build_sc_gather()
    return _sc_gather_built(*args)


def _prep_body(x1_ref, x2_ref, u1_ref, u2_ref, v1_ref, v2_ref,
               j3_ref, sel3_ref, cls2_ref, j4_ref, sel4_ref, cls1_ref,
               diag_ref, pos3_ref, pos4_ref):
    x1 = x1_ref[...]
    x2 = x2_ref[...]
    diag_ref[0, 0, :] = 1.0 - jnp.sum(x1 * x2, axis=1)
    dims = (((1,), (1,)), ((), ()))
    ohc = lax.broadcasted_iota(jnp.int32, (_BS, _NCLS), 1)

    pa = lax.dot_general(x1, u1_ref[...], dims,
                         preferred_element_type=jnp.float32)
    pb = lax.dot_general(x1, u2_ref[...], dims,
                         preferred_element_type=jnp.float32)
    oh2 = cls2_ref[0, 0, :][:, None] == ohc
    p = jnp.where(sel3_ref[0, 0, :] == 1,
                  jnp.sum(jnp.where(oh2, pb, 0.0), axis=1),
                  jnp.sum(jnp.where(oh2, pa, 0.0), axis=1))
    pos3_ref[0, 0, :] = jnp.where(j3_ref[0, 0, :] < _B, 1.0 - p, 0.0)

    qa = lax.dot_general(x2, v1_ref[...], dims,
                         preferred_element_type=jnp.float32)
    qb = lax.dot_general(x2, v2_ref[...], dims,
                         preferred_element_type=jnp.float32)
    oh1 = cls1_ref[0, 0, :][:, None] == ohc
    q = jnp.where(sel4_ref[0, 0, :] == 1,
                  jnp.sum(jnp.where(oh1, qb, 0.0), axis=1),
                  jnp.sum(jnp.where(oh1, qa, 0.0), axis=1))
    pos4_ref[0, 0, :] = jnp.where(j4_ref[0, 0, :] < _B, 1.0 - q, 0.0)


def _prep(x1, x2, u1, u2, v1, v2, j3, sel3, cls2, j4, sel4, cls1):
    mat = pl.BlockSpec((_BS, _D), lambda i: (i, 0))
    rep = pl.BlockSpec((_NCLS, _D), lambda i: (0, 0))
    vec = pl.BlockSpec((1, 1, _BS), lambda i: (i, 0, 0))
    out = jax.ShapeDtypeStruct((_NBLK, 1, _BS), jnp.float32)
    return pl.pallas_call(
        _prep_body,
        grid=(_NBLK,),
        in_specs=[mat, mat, rep, rep, rep, rep, vec, vec, vec, vec, vec, vec],
        out_specs=[vec, vec, vec],
        out_shape=[out] * 3,
    )(x1, x2, u1, u2, v1, v2, j3, sel3, cls2, j4, sel4, cls1)


def _main_body(x1_ref, x2_ref, diagr_ref, diagc_ref, pos3_ref, pos4_ref,
               t3_ref, t4_ref, cls2_ref, cls1_ref,
               s1_ref, n1_ref, s2_ref, n2_ref,
               s3_ref, n3_ref, s4_ref, n4_ref):
    ri = pl.program_id(0)
    ci = pl.program_id(1)
    g2 = (1.0 - _MARGIN) - lax.dot_general(
        x1_ref[...], x2_ref[...], (((1,), (1,)), ((), ())),
        preferred_element_type=jnp.float32)

    io = lax.broadcasted_iota(jnp.int32, (_BR, _BC), 0)
    jo = lax.broadcasted_iota(jnp.int32, (_BR, _BC), 1)
    offd = (ci * _BC - ri * _BR) + jo != io

    def term(keep, v):
        s = jnp.sum(jnp.where(keep, v, 0.0))
        n = jnp.sum(jnp.where(keep & (v > 0), 1.0, 0.0))
        return s, n

    v1 = jnp.maximum(diagr_ref[0, 0, :][:, None] - g2, 0.0)
    s1, n1 = term(offd, v1)
    v2 = jnp.maximum(diagc_ref[0, 0, :][None, :] - g2, 0.0)
    s2, n2 = term(offd, v2)
    anti = cls1_ref[0, 0, :][None, :] != cls2_ref[0, 0, :][:, None]
    th3 = t3_ref[0, 0, :] - ci * _BC
    th4 = t4_ref[0, 0, :] - ri * _BR
    kept3 = anti & (jo < th3[:, None])
    v3 = jnp.maximum(pos3_ref[0, 0, :][:, None] - g2, 0.0)
    s3, n3 = term(kept3, v3)
    kept4 = anti & (io < th4[None, :])
    v4 = jnp.maximum(pos4_ref[0, 0, :][None, :] - g2, 0.0)
    s4, n4 = term(kept4, v4)

    first = (ri == 0) & (ci == 0)
    for ref, val in ((s1_ref, s1), (n1_ref, n1), (s2_ref, s2), (n2_ref, n2),
                     (s3_ref, s3), (n3_ref, n3), (s4_ref, s4), (n4_ref, n4)):
        @pl.when(first)
        def _(ref=ref):
            ref[0, 0] = jnp.zeros((), jnp.float32)
        ref[0, 0] += val


def _main(x1, x2, diag, pos3, pos4, t3, t4, cls2, cls1):
    rmat = pl.BlockSpec((_BR, _D), lambda r, c: (r, 0))
    cmat = pl.BlockSpec((_BC, _D), lambda r, c: (c, 0))
    rvec = pl.BlockSpec((1, 1, _BR), lambda r, c: (r, 0, 0))
    cvec = pl.BlockSpec((1, 1, _BC), lambda r, c: (c, 0, 0))
    acc = pl.BlockSpec(memory_space=pltpu.SMEM)
    out = jax.ShapeDtypeStruct((1, 1), jnp.float32)
    rblk = (_NBR, 1, _BR)
    cblk = (_NBC, 1, _BC)
    return pl.pallas_call(
        _main_body,
        grid=(_NBR, _NBC),
        in_specs=[rmat, cmat, rvec, cvec, rvec, cvec, rvec, cvec, rvec, cvec],
        out_specs=[acc] * 8,
        out_shape=[out] * 8,
        compiler_params=pltpu.CompilerParams(
            dimension_semantics=("arbitrary", "arbitrary")),
    )(x1, x2, diag.reshape(rblk), diag.reshape(cblk),
      pos3.reshape(rblk), pos4.reshape(cblk),
      t3.reshape(rblk), t4.reshape(cblk),
      cls2.reshape(rblk), cls1.reshape(cblk))


def kernel(input1, input2, class1, class2):
    c1 = class1.astype(jnp.int32).reshape(1, _B)
    c2 = class2.astype(jnp.int32).reshape(1, _B)
    (j3, t3, sel3, j4, t4, sel4,
     f1a, f1b, f2a, f2b) = _mining(c1, c2)
    u1, u2, v1, v2 = _sc_gather(input2, input1,
                                f1a.reshape(_NCLS), f1b.reshape(_NCLS),
                                f2a.reshape(_NCLS), f2b.reshape(_NCLS))
    blk = (_NBLK, 1, _BS)
    diag, pos3, pos4 = _prep(input1, input2, u1, u2, v1, v2,
                             j3.reshape(blk), sel3.reshape(blk),
                             c2.reshape(blk),
                             j4.reshape(blk), sel4.reshape(blk),
                             c1.reshape(blk))
    s1, n1, s2, n2, s3, n3, s4, n4 = _main(
        input1, input2, diag, pos3, pos4, t3, t4, c2, c1)
    loss = (s1[0, 0] / n1[0, 0] + s2[0, 0] / n2[0, 0]
            + s3[0, 0] / n3[0, 0] + s4[0, 0] / n4[0, 0])
    return loss.reshape(1)


# speedup vs baseline: 1.0819x; 1.0513x over previous
import functools
import jax
import jax.numpy as jnp
from jax import lax
from jax.experimental import pallas as pl
from jax.experimental.pallas import tpu as pltpu
from jax.experimental.pallas import tpu_sc as plsc

_MARGIN = 0.2
_B = 4096
_D = 1024
_NCLS = 128
_BS = 1024
_NBLK = _B // _BS


def _lane_cumsum(x):
    n = x.shape[1]
    k = 1
    while k < n:
        shifted = jnp.concatenate(
            [jnp.zeros((x.shape[0], k), x.dtype), x[:, : n - k]], axis=1)
        x = x + shifted
        k *= 2
    return x


def _gat(vals, onehot):
    return jnp.sum(jnp.where(onehot, vals, 0), axis=0, keepdims=True)


def _mining_idx_body(c1_ref, c2_ref, f1a_ref, f1b_ref, f2a_ref, f2b_ref):
    c1 = c1_ref[...]
    c2 = c2_ref[...]
    cid = lax.broadcasted_iota(jnp.int32, (_NCLS, _B), 0)
    pos = lax.broadcasted_iota(jnp.int32, (_NCLS, _B), 1)
    eq1 = c1 == cid
    eq2 = c2 == cid
    f1a = jnp.min(jnp.where(eq1, pos, _B), axis=1, keepdims=True)
    f1b = jnp.min(jnp.where(eq1 & (pos != f1a), pos, _B), axis=1, keepdims=True)
    f2a = jnp.min(jnp.where(eq2, pos, _B), axis=1, keepdims=True)
    f2b = jnp.min(jnp.where(eq2 & (pos != f2a), pos, _B), axis=1, keepdims=True)
    f1a_ref[...] = jnp.minimum(f1a, _B - 1)
    f1b_ref[...] = jnp.minimum(f1b, _B - 1)
    f2a_ref[...] = jnp.minimum(f2a, _B - 1)
    f2b_ref[...] = jnp.minimum(f2b, _B - 1)


def _mining_idx(c1, c2):
    col = jax.ShapeDtypeStruct((_NCLS, 1), jnp.int32)
    return pl.pallas_call(
        _mining_idx_body,
        out_shape=[col] * 4,
    )(c1, c2)


def _mining_body(c1_ref, c2_ref,
                 j3_ref, t3_ref, sel3_ref, j4_ref, t4_ref, sel4_ref):
    c1 = c1_ref[...]
    c2 = c2_ref[...]
    cid = lax.broadcasted_iota(jnp.int32, (_NCLS, _B), 0)
    pos = lax.broadcasted_iota(jnp.int32, (_NCLS, _B), 1)
    eq1 = c1 == cid
    eq2 = c2 == cid
    h1 = jnp.sum(eq1.astype(jnp.int32), axis=1, keepdims=True)
    h2 = jnp.sum(eq2.astype(jnp.int32), axis=1, keepdims=True)
    f1a = jnp.min(jnp.where(eq1, pos, _B), axis=1, keepdims=True)
    f1b = jnp.min(jnp.where(eq1 & (pos != f1a), pos, _B), axis=1, keepdims=True)
    f2a = jnp.min(jnp.where(eq2, pos, _B), axis=1, keepdims=True)
    f2b = jnp.min(jnp.where(eq2 & (pos != f2a), pos, _B), axis=1, keepdims=True)
    cum1 = _lane_cumsum(jnp.where(eq1, 0, 1))
    cum2 = _lane_cumsum(jnp.where(eq2, 0, 1))
    rpos = lax.broadcasted_iota(jnp.int32, (1, _B), 1)

    k3 = _B - jnp.max(_gat(h1, eq2)) + 1
    t3class = jnp.min(jnp.where(cum1 == k3, pos, _B), axis=1, keepdims=True)
    t3_ref[...] = _gat(t3class, eq2)
    f1a_at = _gat(f1a, eq2)
    sel3 = f1a_at == rpos
    sel3_ref[...] = sel3.astype(jnp.int32)
    j3_ref[...] = jnp.where(sel3, _gat(f1b, eq2), f1a_at)

    k4 = _B - jnp.max(_gat(h2, eq1)) + 1
    t4class = jnp.min(jnp.where(cum2 == k4, pos, _B), axis=1, keepdims=True)
    t4_ref[...] = _gat(t4class, eq1)
    f2a_at = _gat(f2a, eq1)
    sel4 = f2a_at == rpos
    sel4_ref[...] = sel4.astype(jnp.int32)
    j4_ref[...] = jnp.where(sel4, _gat(f2b, eq1), f2a_at)


def _mining(c1, c2):
    row = jax.ShapeDtypeStruct((1, _B), jnp.int32)
    return pl.pallas_call(
        _mining_body,
        out_shape=[row] * 6,
    )(c1, c2)


_SC_CORES = 2
_SC_SUBCORES = 16
_NW = _SC_CORES * _SC_SUBCORES
_CH = _NCLS // 8


_sc_gather_built = None


def _build_sc_gather():
    @functools.partial(
        pl.kernel,
        mesh=plsc.VectorSubcoreMesh(core_axis_name="c", subcore_axis_name="s",
                                    num_cores=_SC_CORES,
                                    num_subcores=_SC_SUBCORES),
        out_type=[jax.ShapeDtypeStruct((_NCLS, _D), jnp.float32)] * 4,
        scratch_types=[pltpu.VMEM((_CH,), jnp.int32),
                       pltpu.VMEM((_CH, _D), jnp.float32),
                       pltpu.SemaphoreType.DMA],
    )
    def sc_gather(x2_hbm, x1_hbm, f1a_hbm, f1b_hbm, f2a_hbm, f2b_hbm,
                  u1_hbm, u2_hbm, v1_hbm, v2_hbm, idx_v, rows_v, sem):
        wid = lax.axis_index("s") * _SC_CORES + lax.axis_index("c")
        tid = wid // 8
        off = (wid % 8) * _CH
        jobs = ((x2_hbm, f1a_hbm, u1_hbm), (x2_hbm, f1b_hbm, u2_hbm),
                (x1_hbm, f2a_hbm, v1_hbm), (x1_hbm, f2b_hbm, v2_hbm))
        for t, (table, idx_hbm, out) in enumerate(jobs):
            @pl.when(tid == t)
            def _(table=table, idx_hbm=idx_hbm, out=out):
                pltpu.sync_copy(idx_hbm.at[pl.ds(off, _CH)], idx_v)
                pltpu.async_copy(table.at[idx_v], rows_v, sem).wait()
                pltpu.sync_copy(rows_v, out.at[pl.ds(off, _CH)])

    return sc_gather


def _sc_gather(*args):
    global _sc_gather_built
    if _sc_gather_built is None:
        _sc_gather_built = _build_sc_gather()
    return _sc_gather_built(*args)


def _prep_body(x1_ref, x2_ref, u1_ref, u2_ref, v1_ref, v2_ref,
               j3_ref, sel3_ref, cls2_ref, j4_ref, sel4_ref, cls1_ref,
               diag_ref, pos3_ref, pos4_ref):
    x1 = x1_ref[...]
    x2 = x2_ref[...]
    diag_ref[0, 0, :] = 1.0 - jnp.sum(x1 * x2, axis=1)
    dims = (((1,), (1,)), ((), ()))
    ohc = lax.broadcasted_iota(jnp.int32, (_BS, _NCLS), 1)

    pa = lax.dot_general(x1, u1_ref[...], dims,
                         preferred_element_type=jnp.float32)
    pb = lax.dot_general(x1, u2_ref[...], dims,
                         preferred_element_type=jnp.float32)
    oh2 = cls2_ref[0, 0, :][:, None] == ohc
    p = jnp.where(sel3_ref[0, 0, :] == 1,
                  jnp.sum(jnp.where(oh2, pb, 0.0), axis=1),
                  jnp.sum(jnp.where(oh2, pa, 0.0), axis=1))
    pos3_ref[0, 0, :] = jnp.where(j3_ref[0, 0, :] < _B, 1.0 - p, 0.0)

    qa = lax.dot_general(x2, v1_ref[...], dims,
                         preferred_element_type=jnp.float32)
    qb = lax.dot_general(x2, v2_ref[...], dims,
                         preferred_element_type=jnp.float32)
    oh1 = cls1_ref[0, 0, :][:, None] == ohc
    q = jnp.where(sel4_ref[0, 0, :] == 1,
                  jnp.sum(jnp.where(oh1, qb, 0.0), axis=1),
                  jnp.sum(jnp.where(oh1, qa, 0.0), axis=1))
    pos4_ref[0, 0, :] = jnp.where(j4_ref[0, 0, :] < _B, 1.0 - q, 0.0)


def _prep(x1, x2, u1, u2, v1, v2, j3, sel3, cls2, j4, sel4, cls1):
    mat = pl.BlockSpec((_BS, _D), lambda i: (i, 0))
    rep = pl.BlockSpec((_NCLS, _D), lambda i: (0, 0))
    vec = pl.BlockSpec((1, 1, _BS), lambda i: (i, 0, 0))
    out = jax.ShapeDtypeStruct((_NBLK, 1, _BS), jnp.float32)
    return pl.pallas_call(
        _prep_body,
        grid=(_NBLK,),
        in_specs=[mat, mat, rep, rep, rep, rep, vec, vec, vec, vec, vec, vec],
        out_specs=[vec, vec, vec],
        out_shape=[out] * 3,
    )(x1, x2, u1, u2, v1, v2, j3, sel3, cls2, j4, sel4, cls1)


def _main_body(x1_ref, x2_ref, diagr_ref, diagc_ref, pos3_ref, pos4_ref,
               t3_ref, t4_ref, cls2_ref, cls1_ref,
               s1_ref, n1_ref, s2_ref, n2_ref,
               s3_ref, n3_ref, s4_ref, n4_ref):
    ri = pl.program_id(0)
    ci = pl.program_id(1)
    g2 = (1.0 - _MARGIN) - lax.dot_general(
        x1_ref[...], x2_ref[...], (((1,), (1,)), ((), ())),
        preferred_element_type=jnp.float32)

    io = lax.broadcasted_iota(jnp.int32, (_BS, _BS), 0)
    jo = lax.broadcasted_iota(jnp.int32, (_BS, _BS), 1)
    offd = (ri != ci) | (io != jo)

    def term(keep, v):
        s = jnp.sum(jnp.where(keep, v, 0.0))
        n = jnp.sum(jnp.where(keep & (v > 0), 1.0, 0.0))
        return s, n

    v1 = jnp.maximum(diagr_ref[0, 0, :][:, None] - g2, 0.0)
    s1, n1 = term(offd, v1)
    v2 = jnp.maximum(diagc_ref[0, 0, :][None, :] - g2, 0.0)
    s2, n2 = term(offd, v2)
    anti = cls1_ref[0, 0, :][None, :] != cls2_ref[0, 0, :][:, None]
    th3 = t3_ref[0, 0, :] - ci * _BS
    th4 = t4_ref[0, 0, :] - ri * _BS
    kept3 = anti & (jo < th3[:, None])
    v3 = jnp.maximum(pos3_ref[0, 0, :][:, None] - g2, 0.0)
    s3, n3 = term(kept3, v3)
    kept4 = anti & (io < th4[None, :])
    v4 = jnp.maximum(pos4_ref[0, 0, :][None, :] - g2, 0.0)
    s4, n4 = term(kept4, v4)

    first = (ri == 0) & (ci == 0)
    for ref, val in ((s1_ref, s1), (n1_ref, n1), (s2_ref, s2), (n2_ref, n2),
                     (s3_ref, s3), (n3_ref, n3), (s4_ref, s4), (n4_ref, n4)):
        @pl.when(first)
        def _(ref=ref):
            ref[0, 0] = jnp.zeros((), jnp.float32)
        ref[0, 0] += val


def _main(x1, x2, diag, pos3, pos4, t3, t4, cls2, cls1):
    rmat = pl.BlockSpec((_BS, _D), lambda r, c: (r, 0))
    cmat = pl.BlockSpec((_BS, _D), lambda r, c: (c, 0))
    rvec = pl.BlockSpec((1, 1, _BS), lambda r, c: (r, 0, 0))
    cvec = pl.BlockSpec((1, 1, _BS), lambda r, c: (c, 0, 0))
    acc = pl.BlockSpec(memory_space=pltpu.SMEM)
    out = jax.ShapeDtypeStruct((1, 1), jnp.float32)
    return pl.pallas_call(
        _main_body,
        grid=(_NBLK, _NBLK),
        in_specs=[rmat, cmat, rvec, cvec, rvec, cvec, rvec, cvec, rvec, cvec],
        out_specs=[acc] * 8,
        out_shape=[out] * 8,
        compiler_params=pltpu.CompilerParams(
            dimension_semantics=("arbitrary", "arbitrary")),
    )(x1, x2, diag, diag, pos3, pos4, t3, t4, cls2, cls1)


def kernel(input1, input2, class1, class2):
    c1 = class1.astype(jnp.int32).reshape(1, _B)
    c2 = class2.astype(jnp.int32).reshape(1, _B)
    f1a, f1b, f2a, f2b = _mining_idx(c1, c2)
    u1, u2, v1, v2 = _sc_gather(input2, input1,
                                f1a.reshape(_NCLS), f1b.reshape(_NCLS),
                                f2a.reshape(_NCLS), f2b.reshape(_NCLS))
    j3, t3, sel3, j4, t4, sel4 = _mining(c1, c2)
    blk = (_NBLK, 1, _BS)
    diag, pos3, pos4 = _prep(input1, input2, u1, u2, v1, v2,
                             j3.reshape(blk), sel3.reshape(blk),
                             c2.reshape(blk),
                             j4.reshape(blk), sel4.reshape(blk),
                             c1.reshape(blk))
    s1, n1, s2, n2, s3, n3, s4, n4 = _main(
        input1, input2, diag, pos3, pos4,
        t3.reshape(blk), t4.reshape(blk), c2.reshape(blk), c1.reshape(blk))
    loss = (s1[0, 0] / n1[0, 0] + s2[0, 0] / n2[0, 0]
            + s3[0, 0] / n3[0, 0] + s4[0, 0] / n4[0, 0])
    return loss.reshape(1)


# speedup vs baseline: 1.1554x; 1.0680x over previous
import functools
import jax
import jax.numpy as jnp
from jax import lax
from jax.experimental import pallas as pl
from jax.experimental.pallas import tpu as pltpu
from jax.experimental.pallas import tpu_sc as plsc

_MARGIN = 0.2
_B = 4096
_D = 1024
_NCLS = 128
_BS = 1024
_NBLK = _B // _BS


def _lane_cumsum(x):
    n = x.shape[1]
    k = 1
    while k < n:
        shifted = jnp.concatenate(
            [jnp.zeros((x.shape[0], k), x.dtype), x[:, : n - k]], axis=1)
        x = x + shifted
        k *= 2
    return x


def _gat(vals, onehot):
    return jnp.sum(jnp.where(onehot, vals, 0), axis=0, keepdims=True)


def _mining_idx_body(c1_ref, c2_ref, f1a_ref, f1b_ref, f2a_ref, f2b_ref):
    c1 = c1_ref[...]
    c2 = c2_ref[...]
    cid = lax.broadcasted_iota(jnp.int32, (_NCLS, _B), 0)
    pos = lax.broadcasted_iota(jnp.int32, (_NCLS, _B), 1)
    eq1 = c1 == cid
    eq2 = c2 == cid
    f1a = jnp.min(jnp.where(eq1, pos, _B), axis=1, keepdims=True)
    f1b = jnp.min(jnp.where(eq1 & (pos != f1a), pos, _B), axis=1, keepdims=True)
    f2a = jnp.min(jnp.where(eq2, pos, _B), axis=1, keepdims=True)
    f2b = jnp.min(jnp.where(eq2 & (pos != f2a), pos, _B), axis=1, keepdims=True)
    f1a_ref[...] = jnp.minimum(f1a, _B - 1)
    f1b_ref[...] = jnp.minimum(f1b, _B - 1)
    f2a_ref[...] = jnp.minimum(f2a, _B - 1)
    f2b_ref[...] = jnp.minimum(f2b, _B - 1)


def _mining_idx(c1, c2):
    col = jax.ShapeDtypeStruct((_NCLS, 1), jnp.int32)
    return pl.pallas_call(
        _mining_idx_body,
        out_shape=[col] * 4,
    )(c1, c2)


def _mining_body(c1_ref, c2_ref,
                 j3_ref, t3_ref, sel3_ref, j4_ref, t4_ref, sel4_ref):
    c1 = c1_ref[...]
    c2 = c2_ref[...]
    cid = lax.broadcasted_iota(jnp.int32, (_NCLS, _B), 0)
    pos = lax.broadcasted_iota(jnp.int32, (_NCLS, _B), 1)
    eq1 = c1 == cid
    eq2 = c2 == cid
    h1 = jnp.sum(eq1.astype(jnp.int32), axis=1, keepdims=True)
    h2 = jnp.sum(eq2.astype(jnp.int32), axis=1, keepdims=True)
    f1a = jnp.min(jnp.where(eq1, pos, _B), axis=1, keepdims=True)
    f1b = jnp.min(jnp.where(eq1 & (pos != f1a), pos, _B), axis=1, keepdims=True)
    f2a = jnp.min(jnp.where(eq2, pos, _B), axis=1, keepdims=True)
    f2b = jnp.min(jnp.where(eq2 & (pos != f2a), pos, _B), axis=1, keepdims=True)
    cum1 = _lane_cumsum(jnp.where(eq1, 0, 1))
    cum2 = _lane_cumsum(jnp.where(eq2, 0, 1))
    rpos = lax.broadcasted_iota(jnp.int32, (1, _B), 1)

    k3 = _B - jnp.max(_gat(h1, eq2)) + 1
    t3class = jnp.min(jnp.where(cum1 == k3, pos, _B), axis=1, keepdims=True)
    t3_ref[...] = _gat(t3class, eq2)
    f1a_at = _gat(f1a, eq2)
    sel3 = f1a_at == rpos
    sel3_ref[...] = sel3.astype(jnp.int32)
    j3_ref[...] = jnp.where(sel3, _gat(f1b, eq2), f1a_at)

    k4 = _B - jnp.max(_gat(h2, eq1)) + 1
    t4class = jnp.min(jnp.where(cum2 == k4, pos, _B), axis=1, keepdims=True)
    t4_ref[...] = _gat(t4class, eq1)
    f2a_at = _gat(f2a, eq1)
    sel4 = f2a_at == rpos
    sel4_ref[...] = sel4.astype(jnp.int32)
    j4_ref[...] = jnp.where(sel4, _gat(f2b, eq1), f2a_at)


def _mining(c1, c2):
    row = jax.ShapeDtypeStruct((1, _B), jnp.int32)
    return pl.pallas_call(
        _mining_body,
        out_shape=[row] * 6,
    )(c1, c2)


_SC_CORES = 2
_SC_SUBCORES = 16
_NW = _SC_CORES * _SC_SUBCORES
_CH = _NCLS // 8


_sc_gather_built = None


def _build_sc_gather():
    @functools.partial(
        pl.kernel,
        mesh=plsc.VectorSubcoreMesh(core_axis_name="c", subcore_axis_name="s",
                                    num_cores=_SC_CORES,
                                    num_subcores=_SC_SUBCORES),
        out_type=[jax.ShapeDtypeStruct((_NCLS, _D), jnp.float32)] * 4,
        scratch_types=[pltpu.VMEM((_CH,), jnp.int32),
                       pltpu.VMEM((_CH, _D), jnp.float32),
                       pltpu.SemaphoreType.DMA],
    )
    def sc_gather(x2_hbm, x1_hbm, f1a_hbm, f1b_hbm, f2a_hbm, f2b_hbm,
                  u1_hbm, u2_hbm, v1_hbm, v2_hbm, idx_v, rows_v, sem):
        wid = lax.axis_index("s") * _SC_CORES + lax.axis_index("c")
        tid = wid // 8
        off = (wid % 8) * _CH
        jobs = ((x2_hbm, f1a_hbm, u1_hbm), (x2_hbm, f1b_hbm, u2_hbm),
                (x1_hbm, f2a_hbm, v1_hbm), (x1_hbm, f2b_hbm, v2_hbm))
        for t, (table, idx_hbm, out) in enumerate(jobs):
            @pl.when(tid == t)
            def _(table=table, idx_hbm=idx_hbm, out=out):
                pltpu.sync_copy(idx_hbm.at[pl.ds(off, _CH)], idx_v)
                pltpu.async_copy(table.at[idx_v], rows_v, sem).wait()
                pltpu.sync_copy(rows_v, out.at[pl.ds(off, _CH)])

    return sc_gather


def _sc_gather(*args):
    global _sc_gather_built
    if _sc_gather_built is None:
        _sc_gather_built = _build_sc_gather()
    return _sc_gather_built(*args)


def _prep_body(x1_ref, x2_ref, u1_ref, u2_ref, v1_ref, v2_ref,
               j3_ref, sel3_ref, cls2_ref, j4_ref, sel4_ref, cls1_ref,
               diag_ref, pos3_ref, pos4_ref):
    x1 = x1_ref[...]
    x2 = x2_ref[...]
    diag_ref[0, 0, :] = 1.0 - jnp.sum(x1 * x2, axis=1)
    dims = (((1,), (1,)), ((), ()))
    ohc = lax.broadcasted_iota(jnp.int32, (_BS, _NCLS), 1)

    pa = lax.dot_general(x1, u1_ref[...], dims,
                         preferred_element_type=jnp.float32)
    pb = lax.dot_general(x1, u2_ref[...], dims,
                         preferred_element_type=jnp.float32)
    oh2 = cls2_ref[0, 0, :][:, None] == ohc
    p = jnp.where(sel3_ref[0, 0, :] == 1,
                  jnp.sum(jnp.where(oh2, pb, 0.0), axis=1),
                  jnp.sum(jnp.where(oh2, pa, 0.0), axis=1))
    pos3_ref[0, 0, :] = jnp.where(j3_ref[0, 0, :] < _B, 1.0 - p, 0.0)

    qa = lax.dot_general(x2, v1_ref[...], dims,
                         preferred_element_type=jnp.float32)
    qb = lax.dot_general(x2, v2_ref[...], dims,
                         preferred_element_type=jnp.float32)
    oh1 = cls1_ref[0, 0, :][:, None] == ohc
    q = jnp.where(sel4_ref[0, 0, :] == 1,
                  jnp.sum(jnp.where(oh1, qb, 0.0), axis=1),
                  jnp.sum(jnp.where(oh1, qa, 0.0), axis=1))
    pos4_ref[0, 0, :] = jnp.where(j4_ref[0, 0, :] < _B, 1.0 - q, 0.0)


def _prep(x1, x2, u1, u2, v1, v2, j3, sel3, cls2, j4, sel4, cls1):
    mat = pl.BlockSpec((_BS, _D), lambda i: (i, 0))
    rep = pl.BlockSpec((_NCLS, _D), lambda i: (0, 0))
    vec = pl.BlockSpec((1, 1, _BS), lambda i: (i, 0, 0))
    out = jax.ShapeDtypeStruct((_NBLK, 1, _BS), jnp.float32)
    return pl.pallas_call(
        _prep_body,
        grid=(_NBLK,),
        in_specs=[mat, mat, rep, rep, rep, rep, vec, vec, vec, vec, vec, vec],
        out_specs=[vec, vec, vec],
        out_shape=[out] * 3,
    )(x1, x2, u1, u2, v1, v2, j3, sel3, cls2, j4, sel4, cls1)


def _main_body(x1_ref, x2_ref, diagr_ref, diagc_ref, pos3_ref, pos4_ref,
               t3_ref, t4_ref, cls2_ref, cls1_ref,
               s1_ref, n1_ref, s2_ref, n2_ref,
               s3_ref, n3_ref, s4_ref, n4_ref):
    ri = pl.program_id(0)
    ci = pl.program_id(1)
    g2 = (1.0 - _MARGIN) - lax.dot_general(
        x1_ref[...], x2_ref[...], (((1,), (1,)), ((), ())),
        preferred_element_type=jnp.float32)

    io = lax.broadcasted_iota(jnp.int32, (_BS, _BS), 0)
    jo = lax.broadcasted_iota(jnp.int32, (_BS, _BS), 1)
    offd = (ri != ci) | (io != jo)

    def term(keep, v):
        w = jnp.where(keep, v, 0.0)
        s = jnp.sum(w)
        n = jnp.sum(jnp.where(w > 0, 1.0, 0.0))
        return s, n

    v1 = jnp.maximum(diagr_ref[0, 0, :][:, None] - g2, 0.0)
    s1, n1 = term(offd, v1)
    v2 = jnp.maximum(diagc_ref[0, 0, :][None, :] - g2, 0.0)
    s2, n2 = term(offd, v2)
    anti = cls1_ref[0, 0, :][None, :] != cls2_ref[0, 0, :][:, None]
    th3 = t3_ref[0, 0, :] - ci * _BS
    th4 = t4_ref[0, 0, :] - ri * _BS
    kept3 = anti & (jo < th3[:, None])
    v3 = jnp.maximum(pos3_ref[0, 0, :][:, None] - g2, 0.0)
    s3, n3 = term(kept3, v3)
    kept4 = anti & (io < th4[None, :])
    v4 = jnp.maximum(pos4_ref[0, 0, :][None, :] - g2, 0.0)
    s4, n4 = term(kept4, v4)

    first = (ri == 0) & (ci == 0)
    for ref, val in ((s1_ref, s1), (n1_ref, n1), (s2_ref, s2), (n2_ref, n2),
                     (s3_ref, s3), (n3_ref, n3), (s4_ref, s4), (n4_ref, n4)):
        @pl.when(first)
        def _(ref=ref):
            ref[0, 0] = jnp.zeros((), jnp.float32)
        ref[0, 0] += val


def _main(x1, x2, diag, pos3, pos4, t3, t4, cls2, cls1):
    rmat = pl.BlockSpec((_BS, _D), lambda r, c: (r, 0))
    cmat = pl.BlockSpec((_BS, _D), lambda r, c: (c, 0))
    rvec = pl.BlockSpec((1, 1, _BS), lambda r, c: (r, 0, 0))
    cvec = pl.BlockSpec((1, 1, _BS), lambda r, c: (c, 0, 0))
    acc = pl.BlockSpec(memory_space=pltpu.SMEM)
    out = jax.ShapeDtypeStruct((1, 1), jnp.float32)
    return pl.pallas_call(
        _main_body,
        grid=(_NBLK, _NBLK),
        in_specs=[rmat, cmat, rvec, cvec, rvec, cvec, rvec, cvec, rvec, cvec],
        out_specs=[acc] * 8,
        out_shape=[out] * 8,
        compiler_params=pltpu.CompilerParams(
            dimension_semantics=("arbitrary", "arbitrary")),
    )(x1, x2, diag, diag, pos3, pos4, t3, t4, cls2, cls1)


def kernel(input1, input2, class1, class2):
    c1 = class1.astype(jnp.int32).reshape(1, _B)
    c2 = class2.astype(jnp.int32).reshape(1, _B)
    f1a, f1b, f2a, f2b = _mining_idx(c1, c2)
    u1, u2, v1, v2 = _sc_gather(input2, input1,
                                f1a.reshape(_NCLS), f1b.reshape(_NCLS),
                                f2a.reshape(_NCLS), f2b.reshape(_NCLS))
    j3, t3, sel3, j4, t4, sel4 = _mining(c1, c2)
    blk = (_NBLK, 1, _BS)
    diag, pos3, pos4 = _prep(input1, input2, u1, u2, v1, v2,
                             j3.reshape(blk), sel3.reshape(blk),
                             c2.reshape(blk),
                             j4.reshape(blk), sel4.reshape(blk),
                             c1.reshape(blk))
    s1, n1, s2, n2, s3, n3, s4, n4 = _main(
        input1, input2, diag, pos3, pos4,
        t3.reshape(blk), t4.reshape(blk), c2.reshape(blk), c1.reshape(blk))
    loss = (s1[0, 0] / n1[0, 0] + s2[0, 0] / n2[0, 0]
            + s3[0, 0] / n3[0, 0] + s4[0, 0] / n4[0, 0])
    return loss.reshape(1)


# speedup vs baseline: 1.1589x; 1.0031x over previous
import functools
import jax
import jax.numpy as jnp
from jax import lax
from jax.experimental import pallas as pl
from jax.experimental.pallas import tpu as pltpu
from jax.experimental.pallas import tpu_sc as plsc

_MARGIN = 0.2
_B = 4096
_D = 1024
_NCLS = 128
_BS = 1024
_NBLK = _B // _BS


def _lane_cumsum(x):
    n = x.shape[1]
    k = 1
    while k < n:
        shifted = jnp.concatenate(
            [jnp.zeros((x.shape[0], k), x.dtype), x[:, : n - k]], axis=1)
        x = x + shifted
        k *= 2
    return x


def _gat(vals, onehot):
    return jnp.sum(jnp.where(onehot, vals, 0), axis=0, keepdims=True)


def _mining_idx_body(c1_ref, c2_ref, f1a_ref, f1b_ref, f2a_ref, f2b_ref):
    c1 = c1_ref[...]
    c2 = c2_ref[...]
    cid = lax.broadcasted_iota(jnp.int32, (_NCLS, _B), 0)
    pos = lax.broadcasted_iota(jnp.int32, (_NCLS, _B), 1)
    eq1 = c1 == cid
    eq2 = c2 == cid
    f1a = jnp.min(jnp.where(eq1, pos, _B), axis=1, keepdims=True)
    f1b = jnp.min(jnp.where(eq1 & (pos != f1a), pos, _B), axis=1, keepdims=True)
    f2a = jnp.min(jnp.where(eq2, pos, _B), axis=1, keepdims=True)
    f2b = jnp.min(jnp.where(eq2 & (pos != f2a), pos, _B), axis=1, keepdims=True)
    f1a_ref[...] = jnp.minimum(f1a, _B - 1)
    f1b_ref[...] = jnp.minimum(f1b, _B - 1)
    f2a_ref[...] = jnp.minimum(f2a, _B - 1)
    f2b_ref[...] = jnp.minimum(f2b, _B - 1)


def _mining_idx(c1, c2):
    col = jax.ShapeDtypeStruct((_NCLS, 1), jnp.int32)
    return pl.pallas_call(
        _mining_idx_body,
        out_shape=[col] * 4,
    )(c1, c2)


def _mining_body(c1_ref, c2_ref,
                 j3_ref, t3_ref, sel3_ref, j4_ref, t4_ref, sel4_ref):
    c1 = c1_ref[...]
    c2 = c2_ref[...]
    cid = lax.broadcasted_iota(jnp.int32, (_NCLS, _B), 0)
    pos = lax.broadcasted_iota(jnp.int32, (_NCLS, _B), 1)
    eq1 = c1 == cid
    eq2 = c2 == cid
    h1 = jnp.sum(eq1.astype(jnp.int32), axis=1, keepdims=True)
    h2 = jnp.sum(eq2.astype(jnp.int32), axis=1, keepdims=True)
    f1a = jnp.min(jnp.where(eq1, pos, _B), axis=1, keepdims=True)
    f1b = jnp.min(jnp.where(eq1 & (pos != f1a), pos, _B), axis=1, keepdims=True)
    f2a = jnp.min(jnp.where(eq2, pos, _B), axis=1, keepdims=True)
    f2b = jnp.min(jnp.where(eq2 & (pos != f2a), pos, _B), axis=1, keepdims=True)
    cum1 = _lane_cumsum(jnp.where(eq1, 0, 1))
    cum2 = _lane_cumsum(jnp.where(eq2, 0, 1))
    rpos = lax.broadcasted_iota(jnp.int32, (1, _B), 1)

    k3 = _B - jnp.max(_gat(h1, eq2)) + 1
    t3class = jnp.min(jnp.where(cum1 == k3, pos, _B), axis=1, keepdims=True)
    t3_ref[...] = _gat(t3class, eq2)
    f1a_at = _gat(f1a, eq2)
    sel3 = f1a_at == rpos
    sel3_ref[...] = sel3.astype(jnp.int32)
    j3_ref[...] = jnp.where(sel3, _gat(f1b, eq2), f1a_at)

    k4 = _B - jnp.max(_gat(h2, eq1)) + 1
    t4class = jnp.min(jnp.where(cum2 == k4, pos, _B), axis=1, keepdims=True)
    t4_ref[...] = _gat(t4class, eq1)
    f2a_at = _gat(f2a, eq1)
    sel4 = f2a_at == rpos
    sel4_ref[...] = sel4.astype(jnp.int32)
    j4_ref[...] = jnp.where(sel4, _gat(f2b, eq1), f2a_at)


def _mining(c1, c2):
    row = jax.ShapeDtypeStruct((1, _B), jnp.int32)
    return pl.pallas_call(
        _mining_body,
        out_shape=[row] * 6,
    )(c1, c2)


_SC_CORES = 2
_SC_SUBCORES = 16
_NW = _SC_CORES * _SC_SUBCORES
_CH = _NCLS // 8


_sc_gather_built = None


def _build_sc_gather():
    @functools.partial(
        pl.kernel,
        mesh=plsc.VectorSubcoreMesh(core_axis_name="c", subcore_axis_name="s",
                                    num_cores=_SC_CORES,
                                    num_subcores=_SC_SUBCORES),
        out_type=[jax.ShapeDtypeStruct((_NCLS, _D), jnp.float32)] * 4,
        scratch_types=[pltpu.VMEM((_CH,), jnp.int32),
                       pltpu.VMEM((_CH, _D), jnp.float32),
                       pltpu.SemaphoreType.DMA],
    )
    def sc_gather(x2_hbm, x1_hbm, f1a_hbm, f1b_hbm, f2a_hbm, f2b_hbm,
                  u1_hbm, u2_hbm, v1_hbm, v2_hbm, idx_v, rows_v, sem):
        wid = lax.axis_index("s") * _SC_CORES + lax.axis_index("c")
        tid = wid // 8
        off = (wid % 8) * _CH
        jobs = ((x2_hbm, f1a_hbm, u1_hbm), (x2_hbm, f1b_hbm, u2_hbm),
                (x1_hbm, f2a_hbm, v1_hbm), (x1_hbm, f2b_hbm, v2_hbm))
        for t, (table, idx_hbm, out) in enumerate(jobs):
            @pl.when(tid == t)
            def _(table=table, idx_hbm=idx_hbm, out=out):
                pltpu.sync_copy(idx_hbm.at[pl.ds(off, _CH)], idx_v)
                pltpu.async_copy(table.at[idx_v], rows_v, sem).wait()
                pltpu.sync_copy(rows_v, out.at[pl.ds(off, _CH)])

    return sc_gather


def _sc_gather(*args):
    global _sc_gather_built
    if _sc_gather_built is None:
        _sc_gather_built = _build_sc_gather()
    return _sc_gather_built(*args)


def _prep_body(x1_ref, x2_ref, u1_ref, u2_ref, v1_ref, v2_ref,
               j3_ref, sel3_ref, cls2_ref, j4_ref, sel4_ref, cls1_ref,
               diag_ref, pos3_ref, pos4_ref):
    x1 = x1_ref[...]
    x2 = x2_ref[...]
    diag_ref[0, 0, :] = 1.0 - jnp.sum(x1 * x2, axis=1)
    dims = (((1,), (1,)), ((), ()))
    ohc = lax.broadcasted_iota(jnp.int32, (_BS, _NCLS), 1)

    pa = lax.dot_general(x1, u1_ref[...], dims,
                         preferred_element_type=jnp.float32)
    pb = lax.dot_general(x1, u2_ref[...], dims,
                         preferred_element_type=jnp.float32)
    oh2 = cls2_ref[0, 0, :][:, None] == ohc
    p = jnp.where(sel3_ref[0, 0, :] == 1,
                  jnp.sum(jnp.where(oh2, pb, 0.0), axis=1),
                  jnp.sum(jnp.where(oh2, pa, 0.0), axis=1))
    pos3_ref[0, 0, :] = jnp.where(j3_ref[0, 0, :] < _B, 1.0 - p, 0.0)

    qa = lax.dot_general(x2, v1_ref[...], dims,
                         preferred_element_type=jnp.float32)
    qb = lax.dot_general(x2, v2_ref[...], dims,
                         preferred_element_type=jnp.float32)
    oh1 = cls1_ref[0, 0, :][:, None] == ohc
    q = jnp.where(sel4_ref[0, 0, :] == 1,
                  jnp.sum(jnp.where(oh1, qb, 0.0), axis=1),
                  jnp.sum(jnp.where(oh1, qa, 0.0), axis=1))
    pos4_ref[0, 0, :] = jnp.where(j4_ref[0, 0, :] < _B, 1.0 - q, 0.0)


def _prep(x1, x2, u1, u2, v1, v2, j3, sel3, cls2, j4, sel4, cls1):
    mat = pl.BlockSpec((_BS, _D), lambda i: (i, 0))
    rep = pl.BlockSpec((_NCLS, _D), lambda i: (0, 0))
    vec = pl.BlockSpec((1, 1, _BS), lambda i: (i, 0, 0))
    out = jax.ShapeDtypeStruct((_NBLK, 1, _BS), jnp.float32)
    return pl.pallas_call(
        _prep_body,
        grid=(_NBLK,),
        in_specs=[mat, mat, rep, rep, rep, rep, vec, vec, vec, vec, vec, vec],
        out_specs=[vec, vec, vec],
        out_shape=[out] * 3,
    )(x1, x2, u1, u2, v1, v2, j3, sel3, cls2, j4, sel4, cls1)


def _main_body(x1_ref, x2_ref, diagr_ref, diagc_ref, pos3_ref, pos4_ref,
               t3_ref, t4_ref, cls2_ref, cls1_ref,
               s1_ref, n1_ref, s2_ref, n2_ref,
               s3_ref, n3_ref, s4_ref, n4_ref):
    ri = pl.program_id(0)
    ci = pl.program_id(1)
    g2 = (1.0 - _MARGIN) - lax.dot_general(
        x1_ref[...], x2_ref[...], (((1,), (1,)), ((), ())),
        preferred_element_type=jnp.float32)

    io = lax.broadcasted_iota(jnp.int32, (_BS, _BS), 0)
    jo = lax.broadcasted_iota(jnp.int32, (_BS, _BS), 1)

    def term(keep, v):
        w = jnp.where(keep, v, 0.0) if keep is not None else v
        s = jnp.sum(w)
        n = jnp.sum(jnp.where(w > 0, 1.0, 0.0))
        return s, n

    v1 = jnp.maximum(diagr_ref[0, 0, :][:, None] - g2, 0.0)
    s1, n1 = term(None, v1)
    v2 = jnp.maximum(diagc_ref[0, 0, :][None, :] - g2, 0.0)
    s2, n2 = term(None, v2)
    anti = cls1_ref[0, 0, :][None, :] != cls2_ref[0, 0, :][:, None]
    th3 = t3_ref[0, 0, :] - ci * _BS
    th4 = t4_ref[0, 0, :] - ri * _BS
    kept3 = anti & (jo < th3[:, None])
    v3 = jnp.maximum(pos3_ref[0, 0, :][:, None] - g2, 0.0)
    s3, n3 = term(kept3, v3)
    kept4 = anti & (io < th4[None, :])
    v4 = jnp.maximum(pos4_ref[0, 0, :][None, :] - g2, 0.0)
    s4, n4 = term(kept4, v4)

    first = (ri == 0) & (ci == 0)
    for ref, val in ((s1_ref, s1), (n1_ref, n1), (s2_ref, s2), (n2_ref, n2),
                     (s3_ref, s3), (n3_ref, n3), (s4_ref, s4), (n4_ref, n4)):
        @pl.when(first)
        def _(ref=ref):
            ref[0, 0] = jnp.zeros((), jnp.float32)
        ref[0, 0] += val


def _main(x1, x2, diag, pos3, pos4, t3, t4, cls2, cls1):
    rmat = pl.BlockSpec((_BS, _D), lambda r, c: (r, 0))
    cmat = pl.BlockSpec((_BS, _D), lambda r, c: (c, 0))
    rvec = pl.BlockSpec((1, 1, _BS), lambda r, c: (r, 0, 0))
    cvec = pl.BlockSpec((1, 1, _BS), lambda r, c: (c, 0, 0))
    acc = pl.BlockSpec(memory_space=pltpu.SMEM)
    out = jax.ShapeDtypeStruct((1, 1), jnp.float32)
    return pl.pallas_call(
        _main_body,
        grid=(_NBLK, _NBLK),
        in_specs=[rmat, cmat, rvec, cvec, rvec, cvec, rvec, cvec, rvec, cvec],
        out_specs=[acc] * 8,
        out_shape=[out] * 8,
        compiler_params=pltpu.CompilerParams(
            dimension_semantics=("arbitrary", "arbitrary")),
    )(x1, x2, diag, diag, pos3, pos4, t3, t4, cls2, cls1)


def kernel(input1, input2, class1, class2):
    c1 = class1.astype(jnp.int32).reshape(1, _B)
    c2 = class2.astype(jnp.int32).reshape(1, _B)
    f1a, f1b, f2a, f2b = _mining_idx(c1, c2)
    u1, u2, v1, v2 = _sc_gather(input2, input1,
                                f1a.reshape(_NCLS), f1b.reshape(_NCLS),
                                f2a.reshape(_NCLS), f2b.reshape(_NCLS))
    j3, t3, sel3, j4, t4, sel4 = _mining(c1, c2)
    blk = (_NBLK, 1, _BS)
    diag, pos3, pos4 = _prep(input1, input2, u1, u2, v1, v2,
                             j3.reshape(blk), sel3.reshape(blk),
                             c2.reshape(blk),
                             j4.reshape(blk), sel4.reshape(blk),
                             c1.reshape(blk))
    s1, n1, s2, n2, s3, n3, s4, n4 = _main(
        input1, input2, diag, pos3, pos4,
        t3.reshape(blk), t4.reshape(blk), c2.reshape(blk), c1.reshape(blk))
    corr = _MARGIN * _B
    loss = ((s1[0, 0] - corr) / (n1[0, 0] - _B)
            + (s2[0, 0] - corr) / (n2[0, 0] - _B)
            + s3[0, 0] / n3[0, 0] + s4[0, 0] / n4[0, 0])
    return loss.reshape(1)
